# baseline (speedup 1.0000x reference)
import functools

import jax
import jax.numpy as jnp
from jax import lax
from jax.experimental import pallas as pl
from jax.experimental.pallas import tpu as pltpu
from jax.experimental.pallas import tpu_sc as plsc

N = 10000
E = 320000
D = 128
H = 128
P = 128
G = 64

INTERPRET = False

ROW_BLK = 1000
N_BLKS = N // ROW_BLK

NC = 2
NS = 16
NW = NC * NS

CHUNK = 64
NRING = 4
IB = 16
EPAD = -(-E // (NW * IB * CHUNK)) * (NW * IB * CHUNK)
NB0 = EPAD // (NS * IB * CHUNK)
NCH0 = NB0 * IB
NPAD = 10240
ROWS_TILE = NPAD // NS
EH_TILE = E // NW

_vmesh = plsc.VectorSubcoreMesh(core_axis_name="c", subcore_axis_name="s")

import dataclasses as _dc

_sc_cp = pltpu.CompilerParams()
if "needs_layout_passes" in pltpu.CompilerParams.__dataclass_fields__:
    _sc_cp = _dc.replace(_sc_cp, needs_layout_passes=False)


def _dinv_body(hist_ref, o_ref):
    deg = jnp.sum(hist_ref[...], axis=0) + 1.0
    o_ref[...] = lax.rsqrt(deg)[:, None]


def _tc_dinv(hist):
    return pl.pallas_call(
        _dinv_body,
        out_shape=jax.ShapeDtypeStruct((N, 1), jnp.float32),
        interpret=INTERPRET,
    )(hist)


def _in_body(x_ref, win_ref, bin_ref, w1_ref, dinv_ref, h_ref, y_ref):
    h = jnp.dot(x_ref[...], win_ref[...],
                preferred_element_type=jnp.float32) + bin_ref[...]
    h_ref[...] = h
    y_ref[...] = dinv_ref[...] * jnp.dot(h, w1_ref[...],
                                         preferred_element_type=jnp.float32)


def _tc_in(x, W_in, b_in, W1, dinv):
    return pl.pallas_call(
        _in_body,
        grid=(N_BLKS,),
        in_specs=[
            pl.BlockSpec((ROW_BLK, D), lambda j: (j, 0)),
            pl.BlockSpec((D, H), lambda j: (0, 0)),
            pl.BlockSpec((1, H), lambda j: (0, 0)),
            pl.BlockSpec((H, H), lambda j: (0, 0)),
            pl.BlockSpec((ROW_BLK, 1), lambda j: (j, 0)),
        ],
        out_specs=[
            pl.BlockSpec((ROW_BLK, H), lambda j: (j, 0)),
            pl.BlockSpec((ROW_BLK, H), lambda j: (j, 0)),
        ],
        out_shape=[
            jax.ShapeDtypeStruct((N, H), jnp.float32),
            jax.ShapeDtypeStruct((N, H), jnp.float32),
        ],
        interpret=INTERPRET,
    )(x, W_in, b_in.reshape(1, H), W1, dinv)


def _layer_body(res_ref, acc_ref, y_ref, dinv_ref, b_ref, wn_ref,
                r_ref, yn_ref):
    dinv = dinv_ref[...]
    g = dinv * (acc_ref[...] + y_ref[...]) + b_ref[...]
    r = jnp.maximum(res_ref[...] + g, 0.0)
    r_ref[...] = r
    yn_ref[...] = dinv * jnp.dot(r, wn_ref[...],
                                 preferred_element_type=jnp.float32)


def _tc_layer(res, acc, y, dinv, b, W_next):
    return pl.pallas_call(
        _layer_body,
        grid=(N_BLKS,),
        in_specs=[
            pl.BlockSpec((ROW_BLK, H), lambda j: (j, 0)),
            pl.BlockSpec((ROW_BLK, H), lambda j: (j, 0)),
            pl.BlockSpec((ROW_BLK, H), lambda j: (j, 0)),
            pl.BlockSpec((ROW_BLK, 1), lambda j: (j, 0)),
            pl.BlockSpec((1, H), lambda j: (0, 0)),
            pl.BlockSpec((H, H), lambda j: (0, 0)),
        ],
        out_specs=[
            pl.BlockSpec((ROW_BLK, H), lambda j: (j, 0)),
            pl.BlockSpec((ROW_BLK, H), lambda j: (j, 0)),
        ],
        out_shape=[
            jax.ShapeDtypeStruct((N, H), jnp.float32),
            jax.ShapeDtypeStruct((N, H), jnp.float32),
        ],
        interpret=INTERPRET,
    )(res, acc, y, dinv, b.reshape(1, H), W_next)


def _layer3_body(res_ref, acc_ref, y_ref, dinv_ref, b_ref, r_ref):
    g = dinv_ref[...] * (acc_ref[...] + y_ref[...]) + b_ref[...]
    r_ref[...] = jnp.maximum(res_ref[...] + g, 0.0)


def _tc_layer3(res, acc, y, dinv, b):
    return pl.pallas_call(
        _layer3_body,
        grid=(N_BLKS,),
        in_specs=[
            pl.BlockSpec((ROW_BLK, H), lambda j: (j, 0)),
            pl.BlockSpec((ROW_BLK, H), lambda j: (j, 0)),
            pl.BlockSpec((ROW_BLK, H), lambda j: (j, 0)),
            pl.BlockSpec((ROW_BLK, 1), lambda j: (j, 0)),
            pl.BlockSpec((1, H), lambda j: (0, 0)),
        ],
        out_specs=pl.BlockSpec((ROW_BLK, H), lambda j: (j, 0)),
        out_shape=jax.ShapeDtypeStruct((N, H), jnp.float32),
        interpret=INTERPRET,
    )(res, acc, y, dinv, b.reshape(1, H))


def _pool_body(batch_ref, r1_ref, r2_ref, r3_ref, ps_ref, cnt_ref):
    j = pl.program_id(0)
    b = batch_ref[0, 0, :]
    gids = lax.broadcasted_iota(jnp.int32, (G, ROW_BLK), 0)
    mask = (b[None, :] == gids).astype(jnp.float32)

    @pl.when(j == 0)
    def _():
        ps_ref[...] = jnp.zeros_like(ps_ref)
        cnt_ref[...] = jnp.zeros_like(cnt_ref)

    ps_ref[:, 0:H] += jnp.dot(mask, r1_ref[...],
                              preferred_element_type=jnp.float32)
    ps_ref[:, H:2 * H] += jnp.dot(mask, r2_ref[...],
                                  preferred_element_type=jnp.float32)
    ps_ref[:, 2 * H:3 * H] += jnp.dot(mask, r3_ref[...],
                                      preferred_element_type=jnp.float32)
    cnt_ref[...] += jnp.sum(mask, axis=1, keepdims=True)


def _tc_pool(batch3, r1, r2, r3):
    return pl.pallas_call(
        _pool_body,
        grid=(N_BLKS,),
        in_specs=[
            pl.BlockSpec((1, 1, ROW_BLK), lambda j: (j, 0, 0)),
            pl.BlockSpec((ROW_BLK, H), lambda j: (j, 0)),
            pl.BlockSpec((ROW_BLK, H), lambda j: (j, 0)),
            pl.BlockSpec((ROW_BLK, H), lambda j: (j, 0)),
        ],
        out_specs=[
            pl.BlockSpec((G, 3 * H), lambda j: (0, 0)),
            pl.BlockSpec((G, 1), lambda j: (0, 0)),
        ],
        out_shape=[
            jax.ShapeDtypeStruct((G, 3 * H), jnp.float32),
            jax.ShapeDtypeStruct((G, 1), jnp.float32),
        ],
        interpret=INTERPRET,
    )(batch3, r1, r2, r3)


def _head_body(ps_ref, cnt_ref, wp1_ref, bp1_ref, wp2_ref, bp2_ref, o_ref):
    pooled = ps_ref[...] / jnp.maximum(cnt_ref[...], 1.0)
    t = jnp.maximum(jnp.dot(pooled, wp1_ref[...],
                            preferred_element_type=jnp.float32) + bp1_ref[...],
                    0.0)
    p = jnp.dot(t, wp2_ref[...],
                preferred_element_type=jnp.float32) + bp2_ref[...]
    nrm = jnp.sqrt(jnp.sum(p * p, axis=1, keepdims=True))
    o_ref[...] = p / jnp.maximum(nrm, 1e-12)


def _tc_head(ps, cnt, Wp1, bp1, Wp2, bp2):
    return pl.pallas_call(
        _head_body,
        out_shape=jax.ShapeDtypeStruct((G, P), jnp.float32),
        interpret=INTERPRET,
    )(ps, cnt, Wp1, bp1.reshape(1, H), Wp2, bp2.reshape(1, P))


def _sc_hist(dst):
    @functools.partial(
        pl.kernel,
        out_type=jax.ShapeDtypeStruct((NW, N), jnp.float32),
        mesh=_vmesh,
        compiler_params=_sc_cp,
        scratch_types=[
            pltpu.VMEM((EH_TILE,), jnp.int32),
            pltpu.VMEM((N,), jnp.float32),
        ],
    )
    def k(dst_hbm, out_hbm, idxs, hist):
        wid = lax.axis_index("c") * NS + lax.axis_index("s")
        pltpu.sync_copy(dst_hbm.at[pl.ds(wid * EH_TILE, EH_TILE)], idxs)

        zeros = jnp.zeros((16,), jnp.float32)

        @pl.loop(0, N // 16)
        def _(i):
            hist[pl.ds(i * 16, 16)] = zeros

        ones = jnp.ones((16,), jnp.float32)

        @pl.loop(0, EH_TILE // 16)
        def _(i):
            ii = idxs[pl.ds(i * 16, 16)]
            plsc.addupdate_scatter(hist, [ii], ones)

        pltpu.sync_copy(hist, out_hbm.at[wid])

    return k(dst)


def _sc_scatter(y, srcp, dstp):
    @functools.partial(
        pl.kernel,
        out_type=jax.ShapeDtypeStruct((NPAD, H), jnp.float32),
        mesh=_vmesh,
        compiler_params=_sc_cp,
        scratch_types=[
            pltpu.VMEM((IB, CHUNK), jnp.int32),
            pltpu.VMEM((IB, CHUNK), jnp.int32),
            pltpu.VMEM((IB, CHUNK), jnp.int32),
            pltpu.VMEM((IB, CHUNK), jnp.int32),
            pltpu.VMEM((CHUNK, H), jnp.float32),
            pltpu.VMEM((CHUNK, H), jnp.float32),
            pltpu.VMEM((CHUNK, H), jnp.float32),
            pltpu.VMEM((CHUNK, H), jnp.float32),
            pltpu.VMEM_SHARED((NPAD, H), jnp.float32),
            pltpu.SemaphoreType.DMA,
            pltpu.SemaphoreType.DMA,
            pltpu.SemaphoreType.DMA,
        ],
    )
    def k(y_hbm, src_hbm, dst_hbm, out_hbm, s0, d0, s1, d1,
          rowsA, rowsB, rowsC, rowsD, acc, gsem, ssem, isem):
        cid = lax.axis_index("c")
        sid = lax.axis_index("s")
        rows = [rowsA, rowsB, rowsC, rowsD]
        nb_c = NB0
        base_chunk = sid * NCH0

        @pl.when(cid == 0)
        def _():
            zeros = jnp.zeros((16,), jnp.float32)

            @pl.loop(0, CHUNK)
            def _(r):
                @pl.loop(0, H // 16)
                def _(q):
                    rowsA[r, pl.ds(q * 16, 16)] = zeros

            @pl.loop(0, ROWS_TILE // CHUNK)
            def _(z):
                pltpu.sync_copy(
                    rowsA, acc.at[pl.ds(sid * ROWS_TILE + z * CHUNK, CHUNK)])

        plsc.subcore_barrier()

        def i_start(b, sb, db):
            pltpu.async_copy(src_hbm.at[pl.ds(base_chunk + b * IB, IB)],
                             sb, isem)
            pltpu.async_copy(dst_hbm.at[pl.ds(base_chunk + b * IB, IB)],
                             db, isem)

        def i_wait(b, sb, db):
            pltpu.make_async_copy(
                src_hbm.at[pl.ds(base_chunk + b * IB, IB)], sb,
                isem).wait()
            pltpu.make_async_copy(
                dst_hbm.at[pl.ds(base_chunk + b * IB, IB)], db,
                isem).wait()

        def g_start(sb, j, buf):
            pltpu.async_copy(y_hbm.at[sb.at[j]], buf, gsem)

        def g_wait(sb, j, buf):
            pltpu.make_async_copy(y_hbm.at[sb.at[j]], buf, gsem).wait()

        def s_start(db, j, buf):
            pltpu.async_copy(buf, acc.at[db.at[j]], ssem, add=True)

        def s_wait(db, j, buf):
            pltpu.make_async_copy(buf, acc.at[db.at[j]], ssem).wait()

        def do_block(b, sb, db, sb_next, db_next):
            @pl.when(b + 1 < nb_c)
            def _():
                i_start(b + 1, sb_next, db_next)

            for g in range(0, IB, NRING):
                for t in range(NRING):
                    c = g + t
                    g_wait(sb, c, rows[t])
                    s_start(db, c, rows[t])
                if g + NRING < IB:
                    for t in range(NRING):
                        c = g + t
                        s_wait(db, c, rows[t])
                        g_start(sb, g + NRING + t, rows[t])
                else:
                    for t in range(NRING):
                        s_wait(db, g + t, rows[t])

                    @pl.when(b + 1 < nb_c)
                    def _():
                        i_wait(b + 1, sb_next, db_next)
                        for t in range(NRING):
                            g_start(sb_next, t, rows[t])

        @pl.when(cid == 0)
        def _():
            i_start(0, s0, d0)
            i_wait(0, s0, d0)
            for t in range(NRING):
                g_start(s0, t, rows[t])

            @pl.loop(0, nb_c // 2)
            def _(p):
                do_block(2 * p, s0, d0, s1, d1)
                do_block(2 * p + 1, s1, d1, s0, d0)

        plsc.subcore_barrier()

        @pl.when(cid == 0)
        def _():
            pltpu.sync_copy(acc.at[pl.ds(sid * ROWS_TILE, ROWS_TILE)],
                            out_hbm.at[pl.ds(sid * ROWS_TILE, ROWS_TILE)])

    return k(y, srcp, dstp)[:N]


def kernel(x, edge_index, batch, W_in, b_in, W1, b1, W2, b2, W3, b3,
           Wp1, bp1, Wp2, bp2):
    src, dst = edge_index[0], edge_index[1]

    srcp = jnp.concatenate([src, jnp.zeros(EPAD - E, jnp.int32)])
    dstp = jnp.concatenate([dst, jnp.full(EPAD - E, NPAD - 1, jnp.int32)])
    srcp = srcp.reshape(EPAD // CHUNK, CHUNK)
    dstp = dstp.reshape(EPAD // CHUNK, CHUNK)

    hist = _sc_hist(dst)
    dinv = _tc_dinv(hist)

    h, y1 = _tc_in(x, W_in, b_in, W1, dinv)
    a1 = _sc_scatter(y1, srcp, dstp)
    r1, y2 = _tc_layer(h, a1, y1, dinv, b1, W2)
    a2 = _sc_scatter(y2, srcp, dstp)
    r2, y3 = _tc_layer(r1, a2, y2, dinv, b2, W3)
    a3 = _sc_scatter(y3, srcp, dstp)
    r3 = _tc_layer3(r2, a3, y3, dinv, b3)

    batch3 = batch.reshape(N_BLKS, 1, ROW_BLK)
    ps, cnt = _tc_pool(batch3, r1, r2, r3)
    return _tc_head(ps, cnt, Wp1, bp1, Wp2, bp2)


# speedup vs baseline: 1.0528x; 1.0528x over previous
import functools

import jax
import jax.numpy as jnp
from jax import lax
from jax.experimental import pallas as pl
from jax.experimental.pallas import tpu as pltpu
from jax.experimental.pallas import tpu_sc as plsc

N = 10000
E = 320000
D = 128
H = 128
P = 128
G = 64

INTERPRET = False

ROW_BLK = 1000
N_BLKS = N // ROW_BLK

NC = 2
NS = 16
NW = NC * NS

CHUNK = 128
IB = 8
EPAD = -(-E // (NW * IB * CHUNK)) * (NW * IB * CHUNK)
NB = EPAD // (NW * IB * CHUNK)
N_CHUNKS = NB * IB
NPAD = 10240
ROWS_TILE = NPAD // NS
EH_TILE = E // NW

_vmesh = plsc.VectorSubcoreMesh(core_axis_name="c", subcore_axis_name="s")

import dataclasses as _dc

_sc_cp = pltpu.CompilerParams()
if "needs_layout_passes" in pltpu.CompilerParams.__dataclass_fields__:
    _sc_cp = _dc.replace(_sc_cp, needs_layout_passes=False)


def _dinv_body(hist_ref, o_ref):
    deg = jnp.sum(hist_ref[...], axis=0) + 1.0
    o_ref[...] = lax.rsqrt(deg)[:, None]


def _tc_dinv(hist):
    return pl.pallas_call(
        _dinv_body,
        out_shape=jax.ShapeDtypeStruct((N, 1), jnp.float32),
        interpret=INTERPRET,
    )(hist)


def _in_body(x_ref, win_ref, bin_ref, w1_ref, dinv_ref, h_ref, y_ref):
    h = jnp.dot(x_ref[...], win_ref[...],
                preferred_element_type=jnp.float32) + bin_ref[...]
    h_ref[...] = h
    y_ref[...] = dinv_ref[...] * jnp.dot(h, w1_ref[...],
                                         preferred_element_type=jnp.float32)


def _tc_in(x, W_in, b_in, W1, dinv):
    return pl.pallas_call(
        _in_body,
        grid=(N_BLKS,),
        in_specs=[
            pl.BlockSpec((ROW_BLK, D), lambda j: (j, 0)),
            pl.BlockSpec((D, H), lambda j: (0, 0)),
            pl.BlockSpec((1, H), lambda j: (0, 0)),
            pl.BlockSpec((H, H), lambda j: (0, 0)),
            pl.BlockSpec((ROW_BLK, 1), lambda j: (j, 0)),
        ],
        out_specs=[
            pl.BlockSpec((ROW_BLK, H), lambda j: (j, 0)),
            pl.BlockSpec((ROW_BLK, H), lambda j: (j, 0)),
        ],
        out_shape=[
            jax.ShapeDtypeStruct((N, H), jnp.float32),
            jax.ShapeDtypeStruct((N, H), jnp.float32),
        ],
        interpret=INTERPRET,
    )(x, W_in, b_in.reshape(1, H), W1, dinv)


def _layer_body(res_ref, acca_ref, accb_ref, y_ref, dinv_ref, b_ref, wn_ref,
                r_ref, yn_ref):
    dinv = dinv_ref[...]
    g = dinv * (acca_ref[...] + accb_ref[...] + y_ref[...]) + b_ref[...]
    r = jnp.maximum(res_ref[...] + g, 0.0)
    r_ref[...] = r
    yn_ref[...] = dinv * jnp.dot(r, wn_ref[...],
                                 preferred_element_type=jnp.float32)


def _tc_layer(res, acc_a, acc_b, y, dinv, b, W_next):
    return pl.pallas_call(
        _layer_body,
        grid=(N_BLKS,),
        in_specs=[
            pl.BlockSpec((ROW_BLK, H), lambda j: (j, 0)),
            pl.BlockSpec((ROW_BLK, H), lambda j: (j, 0)),
            pl.BlockSpec((ROW_BLK, H), lambda j: (j, 0)),
            pl.BlockSpec((ROW_BLK, H), lambda j: (j, 0)),
            pl.BlockSpec((ROW_BLK, 1), lambda j: (j, 0)),
            pl.BlockSpec((1, H), lambda j: (0, 0)),
            pl.BlockSpec((H, H), lambda j: (0, 0)),
        ],
        out_specs=[
            pl.BlockSpec((ROW_BLK, H), lambda j: (j, 0)),
            pl.BlockSpec((ROW_BLK, H), lambda j: (j, 0)),
        ],
        out_shape=[
            jax.ShapeDtypeStruct((N, H), jnp.float32),
            jax.ShapeDtypeStruct((N, H), jnp.float32),
        ],
        interpret=INTERPRET,
    )(res, acc_a, acc_b, y, dinv, b.reshape(1, H), W_next)


def _layer3_body(res_ref, acca_ref, accb_ref, y_ref, dinv_ref, b_ref, r_ref):
    g = dinv_ref[...] * (acca_ref[...] + accb_ref[...] + y_ref[...]) + b_ref[...]
    r_ref[...] = jnp.maximum(res_ref[...] + g, 0.0)


def _tc_layer3(res, acc_a, acc_b, y, dinv, b):
    return pl.pallas_call(
        _layer3_body,
        grid=(N_BLKS,),
        in_specs=[
            pl.BlockSpec((ROW_BLK, H), lambda j: (j, 0)),
            pl.BlockSpec((ROW_BLK, H), lambda j: (j, 0)),
            pl.BlockSpec((ROW_BLK, H), lambda j: (j, 0)),
            pl.BlockSpec((ROW_BLK, H), lambda j: (j, 0)),
            pl.BlockSpec((ROW_BLK, 1), lambda j: (j, 0)),
            pl.BlockSpec((1, H), lambda j: (0, 0)),
        ],
        out_specs=pl.BlockSpec((ROW_BLK, H), lambda j: (j, 0)),
        out_shape=jax.ShapeDtypeStruct((N, H), jnp.float32),
        interpret=INTERPRET,
    )(res, acc_a, acc_b, y, dinv, b.reshape(1, H))


def _pool_body(batch_ref, r1_ref, r2_ref, r3_ref, ps_ref, cnt_ref):
    j = pl.program_id(0)
    b = batch_ref[0, 0, :]
    gids = lax.broadcasted_iota(jnp.int32, (G, ROW_BLK), 0)
    mask = (b[None, :] == gids).astype(jnp.float32)

    @pl.when(j == 0)
    def _():
        ps_ref[...] = jnp.zeros_like(ps_ref)
        cnt_ref[...] = jnp.zeros_like(cnt_ref)

    ps_ref[:, 0:H] += jnp.dot(mask, r1_ref[...],
                              preferred_element_type=jnp.float32)
    ps_ref[:, H:2 * H] += jnp.dot(mask, r2_ref[...],
                                  preferred_element_type=jnp.float32)
    ps_ref[:, 2 * H:3 * H] += jnp.dot(mask, r3_ref[...],
                                      preferred_element_type=jnp.float32)
    cnt_ref[...] += jnp.sum(mask, axis=1, keepdims=True)


def _tc_pool(batch3, r1, r2, r3):
    return pl.pallas_call(
        _pool_body,
        grid=(N_BLKS,),
        in_specs=[
            pl.BlockSpec((1, 1, ROW_BLK), lambda j: (j, 0, 0)),
            pl.BlockSpec((ROW_BLK, H), lambda j: (j, 0)),
            pl.BlockSpec((ROW_BLK, H), lambda j: (j, 0)),
            pl.BlockSpec((ROW_BLK, H), lambda j: (j, 0)),
        ],
        out_specs=[
            pl.BlockSpec((G, 3 * H), lambda j: (0, 0)),
            pl.BlockSpec((G, 1), lambda j: (0, 0)),
        ],
        out_shape=[
            jax.ShapeDtypeStruct((G, 3 * H), jnp.float32),
            jax.ShapeDtypeStruct((G, 1), jnp.float32),
        ],
        interpret=INTERPRET,
    )(batch3, r1, r2, r3)


def _head_body(ps_ref, cnt_ref, wp1_ref, bp1_ref, wp2_ref, bp2_ref, o_ref):
    pooled = ps_ref[...] / jnp.maximum(cnt_ref[...], 1.0)
    t = jnp.maximum(jnp.dot(pooled, wp1_ref[...],
                            preferred_element_type=jnp.float32) + bp1_ref[...],
                    0.0)
    p = jnp.dot(t, wp2_ref[...],
                preferred_element_type=jnp.float32) + bp2_ref[...]
    nrm = jnp.sqrt(jnp.sum(p * p, axis=1, keepdims=True))
    o_ref[...] = p / jnp.maximum(nrm, 1e-12)


def _tc_head(ps, cnt, Wp1, bp1, Wp2, bp2):
    return pl.pallas_call(
        _head_body,
        out_shape=jax.ShapeDtypeStruct((G, P), jnp.float32),
        interpret=INTERPRET,
    )(ps, cnt, Wp1, bp1.reshape(1, H), Wp2, bp2.reshape(1, P))


def _sc_hist(dst):
    @functools.partial(
        pl.kernel,
        out_type=jax.ShapeDtypeStruct((NW, N), jnp.float32),
        mesh=_vmesh,
        compiler_params=_sc_cp,
        scratch_types=[
            pltpu.VMEM((EH_TILE,), jnp.int32),
            pltpu.VMEM((N,), jnp.float32),
        ],
    )
    def k(dst_hbm, out_hbm, idxs, hist):
        wid = lax.axis_index("c") * NS + lax.axis_index("s")
        pltpu.sync_copy(dst_hbm.at[pl.ds(wid * EH_TILE, EH_TILE)], idxs)

        zeros = jnp.zeros((16,), jnp.float32)

        @pl.loop(0, N // 16)
        def _(i):
            hist[pl.ds(i * 16, 16)] = zeros

        ones = jnp.ones((16,), jnp.float32)

        @pl.loop(0, EH_TILE // 16)
        def _(i):
            ii = idxs[pl.ds(i * 16, 16)]
            plsc.addupdate_scatter(hist, [ii], ones)

        pltpu.sync_copy(hist, out_hbm.at[wid])

    return k(dst)


def _sc_scatter(y, srcp, dstp):
    @functools.partial(
        pl.kernel,
        out_type=jax.ShapeDtypeStruct((NC, NPAD, H), jnp.float32),
        mesh=_vmesh,
        compiler_params=_sc_cp,
        scratch_types=[
            pltpu.VMEM((IB, CHUNK), jnp.int32),
            pltpu.VMEM((IB, CHUNK), jnp.int32),
            pltpu.VMEM((IB, CHUNK), jnp.int32),
            pltpu.VMEM((IB, CHUNK), jnp.int32),
            pltpu.VMEM((CHUNK, H), jnp.float32),
            pltpu.VMEM_SHARED((NPAD, H), jnp.float32),
            pltpu.SemaphoreType.DMA,
            pltpu.SemaphoreType.DMA,
        ],
    )
    def k(y_hbm, src_hbm, dst_hbm, out_hbm, s0, d0, s1, d1,
          rowsA, acc, gsem, isem):
        cid = lax.axis_index("c")
        sid = lax.axis_index("s")
        tile = cid * NS + sid
        base_chunk = tile * N_CHUNKS

        zeros = jnp.zeros((16,), jnp.float32)

        @pl.loop(0, CHUNK)
        def _(r):
            @pl.loop(0, H // 16)
            def _(q):
                rowsA[r, pl.ds(q * 16, 16)] = zeros

        @pl.loop(0, ROWS_TILE // CHUNK)
        def _(z):
            pltpu.sync_copy(
                rowsA, acc.at[pl.ds(sid * ROWS_TILE + z * CHUNK, CHUNK)])

        plsc.subcore_barrier()

        def i_start(b, sb, db):
            pltpu.async_copy(src_hbm.at[pl.ds(base_chunk + b * IB, IB)],
                             sb, isem)
            pltpu.async_copy(dst_hbm.at[pl.ds(base_chunk + b * IB, IB)],
                             db, isem)

        def i_wait(b, sb, db):
            pltpu.make_async_copy(
                src_hbm.at[pl.ds(base_chunk + b * IB, IB)], sb, isem).wait()
            pltpu.make_async_copy(
                dst_hbm.at[pl.ds(base_chunk + b * IB, IB)], db, isem).wait()

        def do_block(b, sb, db, sb_next, db_next):
            @pl.when(b + 1 < NB)
            def _():
                i_start(b + 1, sb_next, db_next)

            for j in range(IB):
                pltpu.async_copy(y_hbm.at[sb.at[j]], rowsA, gsem).wait()
                pltpu.sync_copy(rowsA, acc.at[db.at[j]], add=True)

            @pl.when(b + 1 < NB)
            def _():
                i_wait(b + 1, sb_next, db_next)

        i_start(0, s0, d0)
        i_wait(0, s0, d0)

        @pl.loop(0, NB // 2)
        def _(p):
            do_block(2 * p, s0, d0, s1, d1)
            do_block(2 * p + 1, s1, d1, s0, d0)

        plsc.subcore_barrier()

        pltpu.sync_copy(acc.at[pl.ds(sid * ROWS_TILE, ROWS_TILE)],
                        out_hbm.at[cid, pl.ds(sid * ROWS_TILE, ROWS_TILE)])

    out = k(y, srcp, dstp)
    return out[0, :N], out[1, :N]


def kernel(x, edge_index, batch, W_in, b_in, W1, b1, W2, b2, W3, b3,
           Wp1, bp1, Wp2, bp2):
    src, dst = edge_index[0], edge_index[1]

    srcp = jnp.concatenate([src, jnp.zeros(EPAD - E, jnp.int32)])
    dstp = jnp.concatenate([dst, jnp.full(EPAD - E, NPAD - 1, jnp.int32)])
    srcp = srcp.reshape(EPAD // CHUNK, CHUNK)
    dstp = dstp.reshape(EPAD // CHUNK, CHUNK)

    hist = _sc_hist(dst)
    dinv = _tc_dinv(hist)

    h, y1 = _tc_in(x, W_in, b_in, W1, dinv)
    a1, a1b = _sc_scatter(y1, srcp, dstp)
    r1, y2 = _tc_layer(h, a1, a1b, y1, dinv, b1, W2)
    a2, a2b = _sc_scatter(y2, srcp, dstp)
    r2, y3 = _tc_layer(r1, a2, a2b, y2, dinv, b2, W3)
    a3, a3b = _sc_scatter(y3, srcp, dstp)
    r3 = _tc_layer3(r2, a3, a3b, y3, dinv, b3)

    batch3 = batch.reshape(N_BLKS, 1, ROW_BLK)
    ps, cnt = _tc_pool(batch3, r1, r2, r3)
    return _tc_head(ps, cnt, Wp1, bp1, Wp2, bp2)


# speedup vs baseline: 1.2450x; 1.1826x over previous
import functools

import jax
import jax.numpy as jnp
from jax import lax
from jax.experimental import pallas as pl
from jax.experimental.pallas import tpu as pltpu
from jax.experimental.pallas import tpu_sc as plsc

N = 10000
E = 320000
D = 128
H = 128
P = 128
G = 64

INTERPRET = False

ROW_BLK = 1000
N_BLKS = N // ROW_BLK

NC = 2
NS = 16
NW = NC * NS

CHUNK = 128
EPAD = -(-E // (NW * CHUNK)) * (NW * CHUNK)
TOT_CHUNKS = EPAD // (NS * CHUNK)
NCH0 = 79
NCH1 = TOT_CHUNKS - NCH0
NPAD = 10240
ROWS_TILE = NPAD // NS
EH_TILE = E // NW

_vmesh = plsc.VectorSubcoreMesh(core_axis_name="c", subcore_axis_name="s")

import dataclasses as _dc

_sc_cp = pltpu.CompilerParams()
if "needs_layout_passes" in pltpu.CompilerParams.__dataclass_fields__:
    _sc_cp = _dc.replace(_sc_cp, needs_layout_passes=False)


def _dinv_body(hist_ref, o_ref):
    deg = jnp.sum(hist_ref[...], axis=0) + 1.0
    o_ref[...] = lax.rsqrt(deg)[:, None]


def _tc_dinv(hist):
    return pl.pallas_call(
        _dinv_body,
        out_shape=jax.ShapeDtypeStruct((N, 1), jnp.float32),
        interpret=INTERPRET,
    )(hist)


def _in_body(x_ref, win_ref, bin_ref, w1_ref, dinv_ref, h_ref, y_ref):
    h = jnp.dot(x_ref[...], win_ref[...],
                preferred_element_type=jnp.float32) + bin_ref[...]
    h_ref[...] = h
    y_ref[...] = dinv_ref[...] * jnp.dot(h, w1_ref[...],
                                         preferred_element_type=jnp.float32)


def _tc_in(x, W_in, b_in, W1, dinv):
    return pl.pallas_call(
        _in_body,
        grid=(N_BLKS,),
        in_specs=[
            pl.BlockSpec((ROW_BLK, D), lambda j: (j, 0)),
            pl.BlockSpec((D, H), lambda j: (0, 0)),
            pl.BlockSpec((1, H), lambda j: (0, 0)),
            pl.BlockSpec((H, H), lambda j: (0, 0)),
            pl.BlockSpec((ROW_BLK, 1), lambda j: (j, 0)),
        ],
        out_specs=[
            pl.BlockSpec((ROW_BLK, H), lambda j: (j, 0)),
            pl.BlockSpec((ROW_BLK, H), lambda j: (j, 0)),
        ],
        out_shape=[
            jax.ShapeDtypeStruct((N, H), jnp.float32),
            jax.ShapeDtypeStruct((N, H), jnp.float32),
        ],
        interpret=INTERPRET,
    )(x, W_in, b_in.reshape(1, H), W1, dinv)


def _layer_body(res_ref, acca_ref, accb_ref, y_ref, dinv_ref, b_ref, wn_ref,
                r_ref, yn_ref):
    dinv = dinv_ref[...]
    g = dinv * (acca_ref[...] + accb_ref[...] + y_ref[...]) + b_ref[...]
    r = jnp.maximum(res_ref[...] + g, 0.0)
    r_ref[...] = r
    yn_ref[...] = dinv * jnp.dot(r, wn_ref[...],
                                 preferred_element_type=jnp.float32)


def _tc_layer(res, acc_a, acc_b, y, dinv, b, W_next):
    return pl.pallas_call(
        _layer_body,
        grid=(N_BLKS,),
        in_specs=[
            pl.BlockSpec((ROW_BLK, H), lambda j: (j, 0)),
            pl.BlockSpec((ROW_BLK, H), lambda j: (j, 0)),
            pl.BlockSpec((ROW_BLK, H), lambda j: (j, 0)),
            pl.BlockSpec((ROW_BLK, H), lambda j: (j, 0)),
            pl.BlockSpec((ROW_BLK, 1), lambda j: (j, 0)),
            pl.BlockSpec((1, H), lambda j: (0, 0)),
            pl.BlockSpec((H, H), lambda j: (0, 0)),
        ],
        out_specs=[
            pl.BlockSpec((ROW_BLK, H), lambda j: (j, 0)),
            pl.BlockSpec((ROW_BLK, H), lambda j: (j, 0)),
        ],
        out_shape=[
            jax.ShapeDtypeStruct((N, H), jnp.float32),
            jax.ShapeDtypeStruct((N, H), jnp.float32),
        ],
        interpret=INTERPRET,
    )(res, acc_a, acc_b, y, dinv, b.reshape(1, H), W_next)


def _layer3_body(res_ref, acca_ref, accb_ref, y_ref, dinv_ref, b_ref, r_ref):
    g = dinv_ref[...] * (acca_ref[...] + accb_ref[...] + y_ref[...]) + b_ref[...]
    r_ref[...] = jnp.maximum(res_ref[...] + g, 0.0)


def _tc_layer3(res, acc_a, acc_b, y, dinv, b):
    return pl.pallas_call(
        _layer3_body,
        grid=(N_BLKS,),
        in_specs=[
            pl.BlockSpec((ROW_BLK, H), lambda j: (j, 0)),
            pl.BlockSpec((ROW_BLK, H), lambda j: (j, 0)),
            pl.BlockSpec((ROW_BLK, H), lambda j: (j, 0)),
            pl.BlockSpec((ROW_BLK, H), lambda j: (j, 0)),
            pl.BlockSpec((ROW_BLK, 1), lambda j: (j, 0)),
            pl.BlockSpec((1, H), lambda j: (0, 0)),
        ],
        out_specs=pl.BlockSpec((ROW_BLK, H), lambda j: (j, 0)),
        out_shape=jax.ShapeDtypeStruct((N, H), jnp.float32),
        interpret=INTERPRET,
    )(res, acc_a, acc_b, y, dinv, b.reshape(1, H))


def _pool_body(batch_ref, r1_ref, r2_ref, r3_ref, ps_ref, cnt_ref):
    j = pl.program_id(0)
    b = batch_ref[0, 0, :]
    gids = lax.broadcasted_iota(jnp.int32, (G, ROW_BLK), 0)
    mask = (b[None, :] == gids).astype(jnp.float32)

    @pl.when(j == 0)
    def _():
        ps_ref[...] = jnp.zeros_like(ps_ref)
        cnt_ref[...] = jnp.zeros_like(cnt_ref)

    ps_ref[:, 0:H] += jnp.dot(mask, r1_ref[...],
                              preferred_element_type=jnp.float32)
    ps_ref[:, H:2 * H] += jnp.dot(mask, r2_ref[...],
                                  preferred_element_type=jnp.float32)
    ps_ref[:, 2 * H:3 * H] += jnp.dot(mask, r3_ref[...],
                                      preferred_element_type=jnp.float32)
    cnt_ref[...] += jnp.sum(mask, axis=1, keepdims=True)


def _tc_pool(batch3, r1, r2, r3):
    return pl.pallas_call(
        _pool_body,
        grid=(N_BLKS,),
        in_specs=[
            pl.BlockSpec((1, 1, ROW_BLK), lambda j: (j, 0, 0)),
            pl.BlockSpec((ROW_BLK, H), lambda j: (j, 0)),
            pl.BlockSpec((ROW_BLK, H), lambda j: (j, 0)),
            pl.BlockSpec((ROW_BLK, H), lambda j: (j, 0)),
        ],
        out_specs=[
            pl.BlockSpec((G, 3 * H), lambda j: (0, 0)),
            pl.BlockSpec((G, 1), lambda j: (0, 0)),
        ],
        out_shape=[
            jax.ShapeDtypeStruct((G, 3 * H), jnp.float32),
            jax.ShapeDtypeStruct((G, 1), jnp.float32),
        ],
        interpret=INTERPRET,
    )(batch3, r1, r2, r3)


def _head_body(ps_ref, cnt_ref, wp1_ref, bp1_ref, wp2_ref, bp2_ref, o_ref):
    pooled = ps_ref[...] / jnp.maximum(cnt_ref[...], 1.0)
    t = jnp.maximum(jnp.dot(pooled, wp1_ref[...],
                            preferred_element_type=jnp.float32) + bp1_ref[...],
                    0.0)
    p = jnp.dot(t, wp2_ref[...],
                preferred_element_type=jnp.float32) + bp2_ref[...]
    nrm = jnp.sqrt(jnp.sum(p * p, axis=1, keepdims=True))
    o_ref[...] = p / jnp.maximum(nrm, 1e-12)


def _tc_head(ps, cnt, Wp1, bp1, Wp2, bp2):
    return pl.pallas_call(
        _head_body,
        out_shape=jax.ShapeDtypeStruct((G, P), jnp.float32),
        interpret=INTERPRET,
    )(ps, cnt, Wp1, bp1.reshape(1, H), Wp2, bp2.reshape(1, P))


def _sc_hist(dst):
    @functools.partial(
        pl.kernel,
        out_type=jax.ShapeDtypeStruct((NW, N), jnp.float32),
        mesh=_vmesh,
        compiler_params=_sc_cp,
        scratch_types=[
            pltpu.VMEM((EH_TILE,), jnp.int32),
            pltpu.VMEM((N,), jnp.float32),
        ],
    )
    def k(dst_hbm, out_hbm, idxs, hist):
        wid = lax.axis_index("c") * NS + lax.axis_index("s")
        pltpu.sync_copy(dst_hbm.at[pl.ds(wid * EH_TILE, EH_TILE)], idxs)

        zeros = jnp.zeros((16,), jnp.float32)

        @pl.loop(0, N // 16)
        def _(i):
            hist[pl.ds(i * 16, 16)] = zeros

        ones = jnp.ones((16,), jnp.float32)

        @pl.loop(0, EH_TILE // 16)
        def _(i):
            ii = idxs[pl.ds(i * 16, 16)]
            plsc.addupdate_scatter(hist, [ii], ones)

        pltpu.sync_copy(hist, out_hbm.at[wid])

    return k(dst)


def _sc_scatter(y, srcp, dstp):
    @functools.partial(
        pl.kernel,
        out_type=jax.ShapeDtypeStruct((NC, NPAD, H), jnp.float32),
        mesh=_vmesh,
        compiler_params=_sc_cp,
        scratch_types=[
            pltpu.VMEM((CHUNK,), jnp.int32),
            pltpu.VMEM((CHUNK,), jnp.int32),
            pltpu.VMEM((CHUNK, H), jnp.float32),
            pltpu.VMEM_SHARED((NPAD, H), jnp.float32),
            pltpu.SemaphoreType.DMA,
        ],
    )
    def k(y_hbm, src_hbm, dst_hbm, out_hbm, sidx, didx, rows, acc, sem):
        cid = lax.axis_index("c")
        sid = lax.axis_index("s")
        nch_c = jnp.where(cid == 0, NCH0, NCH1)
        base = jnp.where(cid == 0, sid * NCH0, NS * NCH0 + sid * NCH1) * CHUNK

        zeros = jnp.zeros((16,), jnp.float32)

        @pl.loop(0, CHUNK)
        def _(r):
            @pl.loop(0, H // 16)
            def _(q):
                rows[r, pl.ds(q * 16, 16)] = zeros

        @pl.loop(0, ROWS_TILE // CHUNK)
        def _(z):
            pltpu.sync_copy(
                rows, acc.at[pl.ds(sid * ROWS_TILE + z * CHUNK, CHUNK)])

        plsc.subcore_barrier()

        @pl.loop(0, nch_c)
        def _(cnk):
            off = base + cnk * CHUNK
            pltpu.sync_copy(src_hbm.at[pl.ds(off, CHUNK)], sidx)
            pltpu.sync_copy(dst_hbm.at[pl.ds(off, CHUNK)], didx)
            pltpu.async_copy(y_hbm.at[sidx], rows, sem).wait()
            pltpu.sync_copy(rows, acc.at[didx], add=True)

        plsc.subcore_barrier()

        pltpu.sync_copy(acc.at[pl.ds(sid * ROWS_TILE, ROWS_TILE)],
                        out_hbm.at[cid, pl.ds(sid * ROWS_TILE, ROWS_TILE)])

    out = k(y, srcp, dstp)
    return out[0, :N], out[1, :N]


def kernel(x, edge_index, batch, W_in, b_in, W1, b1, W2, b2, W3, b3,
           Wp1, bp1, Wp2, bp2):
    src, dst = edge_index[0], edge_index[1]

    srcp = jnp.concatenate([src, jnp.zeros(EPAD - E, jnp.int32)])
    dstp = jnp.concatenate([dst, jnp.full(EPAD - E, NPAD - 1, jnp.int32)])

    hist = _sc_hist(dst)
    dinv = _tc_dinv(hist)

    h, y1 = _tc_in(x, W_in, b_in, W1, dinv)
    a1, a1b = _sc_scatter(y1, srcp, dstp)
    r1, y2 = _tc_layer(h, a1, a1b, y1, dinv, b1, W2)
    a2, a2b = _sc_scatter(y2, srcp, dstp)
    r2, y3 = _tc_layer(r1, a2, a2b, y2, dinv, b2, W3)
    a3, a3b = _sc_scatter(y3, srcp, dstp)
    r3 = _tc_layer3(r2, a3, a3b, y3, dinv, b3)

    batch3 = batch.reshape(N_BLKS, 1, ROW_BLK)
    ps, cnt = _tc_pool(batch3, r1, r2, r3)
    return _tc_head(ps, cnt, Wp1, bp1, Wp2, bp2)


# speedup vs baseline: 1.4135x; 1.1354x over previous
import functools

import jax
import jax.numpy as jnp
from jax import lax
from jax.experimental import pallas as pl
from jax.experimental.pallas import tpu as pltpu
from jax.experimental.pallas import tpu_sc as plsc

N = 10000
E = 320000
D = 128
H = 128
P = 128
G = 64

INTERPRET = False

ROW_BLK = 1000
N_BLKS = N // ROW_BLK

NC = 2
NS = 16
NW = NC * NS

CHUNK = 128
EPAD = -(-E // (NW * CHUNK)) * (NW * CHUNK)
TOT_CHUNKS = EPAD // (NS * CHUNK)
NCH0 = 99
NCH1 = TOT_CHUNKS - NCH0
NPAD = 10240
ROWS_TILE = NPAD // NS
EH_TILE = E // NW

_vmesh = plsc.VectorSubcoreMesh(core_axis_name="c", subcore_axis_name="s")

import dataclasses as _dc

_sc_cp = pltpu.CompilerParams()
if "needs_layout_passes" in pltpu.CompilerParams.__dataclass_fields__:
    _sc_cp = _dc.replace(_sc_cp, needs_layout_passes=False)


def _dinv_body(hist_ref, o_ref):
    deg = jnp.sum(hist_ref[...], axis=0) + 1.0
    o_ref[...] = lax.rsqrt(deg)[:, None]


def _tc_dinv(hist):
    return pl.pallas_call(
        _dinv_body,
        out_shape=jax.ShapeDtypeStruct((N, 1), jnp.float32),
        interpret=INTERPRET,
    )(hist)


def _in_body(x_ref, win_ref, bin_ref, w1_ref, dinv_ref, h_ref, y_ref):
    h = jnp.dot(x_ref[...], win_ref[...],
                preferred_element_type=jnp.float32) + bin_ref[...]
    h_ref[...] = h
    y_ref[...] = dinv_ref[...] * jnp.dot(h, w1_ref[...],
                                         preferred_element_type=jnp.float32)


def _tc_in(x, W_in, b_in, W1, dinv):
    return pl.pallas_call(
        _in_body,
        grid=(N_BLKS,),
        in_specs=[
            pl.BlockSpec((ROW_BLK, D), lambda j: (j, 0)),
            pl.BlockSpec((D, H), lambda j: (0, 0)),
            pl.BlockSpec((1, H), lambda j: (0, 0)),
            pl.BlockSpec((H, H), lambda j: (0, 0)),
            pl.BlockSpec((ROW_BLK, 1), lambda j: (j, 0)),
        ],
        out_specs=[
            pl.BlockSpec((ROW_BLK, H), lambda j: (j, 0)),
            pl.BlockSpec((ROW_BLK, H), lambda j: (j, 0)),
        ],
        out_shape=[
            jax.ShapeDtypeStruct((N, H), jnp.float32),
            jax.ShapeDtypeStruct((N, H), jnp.float32),
        ],
        interpret=INTERPRET,
    )(x, W_in, b_in.reshape(1, H), W1, dinv)


def _layer_body(res_ref, acca_ref, accb_ref, y_ref, dinv_ref, b_ref, wn_ref,
                r_ref, yn_ref):
    dinv = dinv_ref[...]
    g = dinv * (acca_ref[...] + accb_ref[...] + y_ref[...]) + b_ref[...]
    r = jnp.maximum(res_ref[...] + g, 0.0)
    r_ref[...] = r
    yn_ref[...] = dinv * jnp.dot(r, wn_ref[...],
                                 preferred_element_type=jnp.float32)


def _tc_layer(res, acc_a, acc_b, y, dinv, b, W_next):
    return pl.pallas_call(
        _layer_body,
        grid=(N_BLKS,),
        in_specs=[
            pl.BlockSpec((ROW_BLK, H), lambda j: (j, 0)),
            pl.BlockSpec((ROW_BLK, H), lambda j: (j, 0)),
            pl.BlockSpec((ROW_BLK, H), lambda j: (j, 0)),
            pl.BlockSpec((ROW_BLK, H), lambda j: (j, 0)),
            pl.BlockSpec((ROW_BLK, 1), lambda j: (j, 0)),
            pl.BlockSpec((1, H), lambda j: (0, 0)),
            pl.BlockSpec((H, H), lambda j: (0, 0)),
        ],
        out_specs=[
            pl.BlockSpec((ROW_BLK, H), lambda j: (j, 0)),
            pl.BlockSpec((ROW_BLK, H), lambda j: (j, 0)),
        ],
        out_shape=[
            jax.ShapeDtypeStruct((N, H), jnp.float32),
            jax.ShapeDtypeStruct((N, H), jnp.float32),
        ],
        interpret=INTERPRET,
    )(res, acc_a, acc_b, y, dinv, b.reshape(1, H), W_next)


def _layer3_body(res_ref, acca_ref, accb_ref, y_ref, dinv_ref, b_ref, r_ref):
    g = dinv_ref[...] * (acca_ref[...] + accb_ref[...] + y_ref[...]) + b_ref[...]
    r_ref[...] = jnp.maximum(res_ref[...] + g, 0.0)


def _tc_layer3(res, acc_a, acc_b, y, dinv, b):
    return pl.pallas_call(
        _layer3_body,
        grid=(N_BLKS,),
        in_specs=[
            pl.BlockSpec((ROW_BLK, H), lambda j: (j, 0)),
            pl.BlockSpec((ROW_BLK, H), lambda j: (j, 0)),
            pl.BlockSpec((ROW_BLK, H), lambda j: (j, 0)),
            pl.BlockSpec((ROW_BLK, H), lambda j: (j, 0)),
            pl.BlockSpec((ROW_BLK, 1), lambda j: (j, 0)),
            pl.BlockSpec((1, H), lambda j: (0, 0)),
        ],
        out_specs=pl.BlockSpec((ROW_BLK, H), lambda j: (j, 0)),
        out_shape=jax.ShapeDtypeStruct((N, H), jnp.float32),
        interpret=INTERPRET,
    )(res, acc_a, acc_b, y, dinv, b.reshape(1, H))


def _pool_body(batch_ref, r1_ref, r2_ref, r3_ref, ps_ref, cnt_ref):
    j = pl.program_id(0)
    b = batch_ref[0, 0, :]
    gids = lax.broadcasted_iota(jnp.int32, (G, ROW_BLK), 0)
    mask = (b[None, :] == gids).astype(jnp.float32)

    @pl.when(j == 0)
    def _():
        ps_ref[...] = jnp.zeros_like(ps_ref)
        cnt_ref[...] = jnp.zeros_like(cnt_ref)

    ps_ref[:, 0:H] += jnp.dot(mask, r1_ref[...],
                              preferred_element_type=jnp.float32)
    ps_ref[:, H:2 * H] += jnp.dot(mask, r2_ref[...],
                                  preferred_element_type=jnp.float32)
    ps_ref[:, 2 * H:3 * H] += jnp.dot(mask, r3_ref[...],
                                      preferred_element_type=jnp.float32)
    cnt_ref[...] += jnp.sum(mask, axis=1, keepdims=True)


def _tc_pool(batch3, r1, r2, r3):
    return pl.pallas_call(
        _pool_body,
        grid=(N_BLKS,),
        in_specs=[
            pl.BlockSpec((1, 1, ROW_BLK), lambda j: (j, 0, 0)),
            pl.BlockSpec((ROW_BLK, H), lambda j: (j, 0)),
            pl.BlockSpec((ROW_BLK, H), lambda j: (j, 0)),
            pl.BlockSpec((ROW_BLK, H), lambda j: (j, 0)),
        ],
        out_specs=[
            pl.BlockSpec((G, 3 * H), lambda j: (0, 0)),
            pl.BlockSpec((G, 1), lambda j: (0, 0)),
        ],
        out_shape=[
            jax.ShapeDtypeStruct((G, 3 * H), jnp.float32),
            jax.ShapeDtypeStruct((G, 1), jnp.float32),
        ],
        interpret=INTERPRET,
    )(batch3, r1, r2, r3)


def _head_body(ps_ref, cnt_ref, wp1_ref, bp1_ref, wp2_ref, bp2_ref, o_ref):
    pooled = ps_ref[...] / jnp.maximum(cnt_ref[...], 1.0)
    t = jnp.maximum(jnp.dot(pooled, wp1_ref[...],
                            preferred_element_type=jnp.float32) + bp1_ref[...],
                    0.0)
    p = jnp.dot(t, wp2_ref[...],
                preferred_element_type=jnp.float32) + bp2_ref[...]
    nrm = jnp.sqrt(jnp.sum(p * p, axis=1, keepdims=True))
    o_ref[...] = p / jnp.maximum(nrm, 1e-12)


def _tc_head(ps, cnt, Wp1, bp1, Wp2, bp2):
    return pl.pallas_call(
        _head_body,
        out_shape=jax.ShapeDtypeStruct((G, P), jnp.float32),
        interpret=INTERPRET,
    )(ps, cnt, Wp1, bp1.reshape(1, H), Wp2, bp2.reshape(1, P))


def _sc_hist(dst):
    @functools.partial(
        pl.kernel,
        out_type=jax.ShapeDtypeStruct((NW, N), jnp.float32),
        mesh=_vmesh,
        compiler_params=_sc_cp,
        scratch_types=[
            pltpu.VMEM((EH_TILE,), jnp.int32),
            pltpu.VMEM((N,), jnp.float32),
        ],
    )
    def k(dst_hbm, out_hbm, idxs, hist):
        wid = lax.axis_index("c") * NS + lax.axis_index("s")
        pltpu.sync_copy(dst_hbm.at[pl.ds(wid * EH_TILE, EH_TILE)], idxs)

        zeros = jnp.zeros((16,), jnp.float32)

        @pl.loop(0, N // 16)
        def _(i):
            hist[pl.ds(i * 16, 16)] = zeros

        ones = jnp.ones((16,), jnp.float32)

        @pl.loop(0, EH_TILE // 16)
        def _(i):
            ii = idxs[pl.ds(i * 16, 16)]
            plsc.addupdate_scatter(hist, [ii], ones)

        pltpu.sync_copy(hist, out_hbm.at[wid])

    return k(dst)


def _sc_scatter(y, srcp, dstp):
    @functools.partial(
        pl.kernel,
        out_type=jax.ShapeDtypeStruct((NC, NPAD, H), jnp.float32),
        mesh=_vmesh,
        compiler_params=_sc_cp,
        scratch_types=[
            pltpu.VMEM((CHUNK,), jnp.int32),
            pltpu.VMEM((CHUNK,), jnp.int32),
            pltpu.VMEM((CHUNK, H), jnp.float32),
            pltpu.VMEM_SHARED((NPAD, H), jnp.float32),
            pltpu.SemaphoreType.DMA,
        ],
    )
    def k(y_hbm, src_hbm, dst_hbm, out_hbm, sidx, didx, rows, acc, sem):
        cid = lax.axis_index("c")
        sid = lax.axis_index("s")
        nch_c = jnp.where(cid == 0, NCH0, NCH1)
        base = jnp.where(cid == 0, sid * NCH0, NS * NCH0 + sid * NCH1) * CHUNK

        zeros = jnp.zeros((16,), jnp.float32)

        @pl.loop(0, CHUNK)
        def _(r):
            @pl.loop(0, H // 16)
            def _(q):
                rows[r, pl.ds(q * 16, 16)] = zeros

        @pl.loop(0, ROWS_TILE // CHUNK)
        def _(z):
            pltpu.sync_copy(
                rows, acc.at[pl.ds(sid * ROWS_TILE + z * CHUNK, CHUNK)])

        plsc.subcore_barrier()

        @pl.loop(0, nch_c)
        def _(cnk):
            off = base + cnk * CHUNK
            pltpu.sync_copy(src_hbm.at[pl.ds(off, CHUNK)], sidx)
            pltpu.sync_copy(dst_hbm.at[pl.ds(off, CHUNK)], didx)
            pltpu.async_copy(y_hbm.at[sidx], rows, sem).wait()
            pltpu.sync_copy(rows, acc.at[didx], add=True)

        plsc.subcore_barrier()

        pltpu.sync_copy(acc.at[pl.ds(sid * ROWS_TILE, ROWS_TILE)],
                        out_hbm.at[cid, pl.ds(sid * ROWS_TILE, ROWS_TILE)])

    out = k(y, srcp, dstp)
    return out[0, :N], out[1, :N]


def kernel(x, edge_index, batch, W_in, b_in, W1, b1, W2, b2, W3, b3,
           Wp1, bp1, Wp2, bp2):
    src, dst = edge_index[0], edge_index[1]

    srcp = jnp.concatenate([src, jnp.zeros(EPAD - E, jnp.int32)])
    dstp = jnp.concatenate([dst, jnp.full(EPAD - E, NPAD - 1, jnp.int32)])

    hist = _sc_hist(dst)
    dinv = _tc_dinv(hist)

    h, y1 = _tc_in(x, W_in, b_in, W1, dinv)
    a1, a1b = _sc_scatter(y1, srcp, dstp)
    r1, y2 = _tc_layer(h, a1, a1b, y1, dinv, b1, W2)
    a2, a2b = _sc_scatter(y2, srcp, dstp)
    r2, y3 = _tc_layer(r1, a2, a2b, y2, dinv, b2, W3)
    a3, a3b = _sc_scatter(y3, srcp, dstp)
    r3 = _tc_layer3(r2, a3, a3b, y3, dinv, b3)

    batch3 = batch.reshape(N_BLKS, 1, ROW_BLK)
    ps, cnt = _tc_pool(batch3, r1, r2, r3)
    return _tc_head(ps, cnt, Wp1, bp1, Wp2, bp2)


# speedup vs baseline: 1.8009x; 1.2740x over previous
import functools

import jax
import jax.numpy as jnp
from jax import lax
from jax.experimental import pallas as pl
from jax.experimental.pallas import tpu as pltpu
from jax.experimental.pallas import tpu_sc as plsc

N = 10000
E = 320000
D = 128
H = 128
P = 128
G = 64

INTERPRET = False

ROW_BLK = 1000
N_BLKS = N // ROW_BLK

NC = 2
NS = 16
NW = NC * NS

CHUNK = 128
IB = 8
EPAD = -(-E // (NW * CHUNK)) * (NW * CHUNK)
TOT_CHUNKS = EPAD // (NS * CHUNK)
NCH0 = 128
NB0 = NCH0 // IB
NCH1 = TOT_CHUNKS - NCH0
NPAD = 10240
ROWS_TILE = NPAD // NS
EH_TILE = E // NW

_vmesh = plsc.VectorSubcoreMesh(core_axis_name="c", subcore_axis_name="s")

import dataclasses as _dc

_sc_cp = pltpu.CompilerParams()
if "needs_layout_passes" in pltpu.CompilerParams.__dataclass_fields__:
    _sc_cp = _dc.replace(_sc_cp, needs_layout_passes=False)


def _dinv_body(hist_ref, o_ref):
    deg = jnp.sum(hist_ref[...], axis=0) + 1.0
    o_ref[...] = lax.rsqrt(deg)[:, None]


def _tc_dinv(hist):
    return pl.pallas_call(
        _dinv_body,
        out_shape=jax.ShapeDtypeStruct((N, 1), jnp.float32),
        interpret=INTERPRET,
    )(hist)


def _in_body(x_ref, win_ref, bin_ref, w1_ref, dinv_ref, h_ref, y_ref):
    h = jnp.dot(x_ref[...], win_ref[...],
                preferred_element_type=jnp.float32) + bin_ref[...]
    h_ref[...] = h
    y_ref[...] = dinv_ref[...] * jnp.dot(h, w1_ref[...],
                                         preferred_element_type=jnp.float32)


def _tc_in(x, W_in, b_in, W1, dinv):
    return pl.pallas_call(
        _in_body,
        grid=(N_BLKS,),
        in_specs=[
            pl.BlockSpec((ROW_BLK, D), lambda j: (j, 0)),
            pl.BlockSpec((D, H), lambda j: (0, 0)),
            pl.BlockSpec((1, H), lambda j: (0, 0)),
            pl.BlockSpec((H, H), lambda j: (0, 0)),
            pl.BlockSpec((ROW_BLK, 1), lambda j: (j, 0)),
        ],
        out_specs=[
            pl.BlockSpec((ROW_BLK, H), lambda j: (j, 0)),
            pl.BlockSpec((ROW_BLK, H), lambda j: (j, 0)),
        ],
        out_shape=[
            jax.ShapeDtypeStruct((N, H), jnp.float32),
            jax.ShapeDtypeStruct((N, H), jnp.float32),
        ],
        interpret=INTERPRET,
    )(x, W_in, b_in.reshape(1, H), W1, dinv)


def _layer_body(res_ref, acca_ref, accb_ref, y_ref, dinv_ref, b_ref, wn_ref,
                r_ref, yn_ref):
    dinv = dinv_ref[...]
    g = dinv * (acca_ref[...] + accb_ref[...] + y_ref[...]) + b_ref[...]
    r = jnp.maximum(res_ref[...] + g, 0.0)
    r_ref[...] = r
    yn_ref[...] = dinv * jnp.dot(r, wn_ref[...],
                                 preferred_element_type=jnp.float32)


def _tc_layer(res, acc_a, acc_b, y, dinv, b, W_next):
    return pl.pallas_call(
        _layer_body,
        grid=(N_BLKS,),
        in_specs=[
            pl.BlockSpec((ROW_BLK, H), lambda j: (j, 0)),
            pl.BlockSpec((ROW_BLK, H), lambda j: (j, 0)),
            pl.BlockSpec((ROW_BLK, H), lambda j: (j, 0)),
            pl.BlockSpec((ROW_BLK, H), lambda j: (j, 0)),
            pl.BlockSpec((ROW_BLK, 1), lambda j: (j, 0)),
            pl.BlockSpec((1, H), lambda j: (0, 0)),
            pl.BlockSpec((H, H), lambda j: (0, 0)),
        ],
        out_specs=[
            pl.BlockSpec((ROW_BLK, H), lambda j: (j, 0)),
            pl.BlockSpec((ROW_BLK, H), lambda j: (j, 0)),
        ],
        out_shape=[
            jax.ShapeDtypeStruct((N, H), jnp.float32),
            jax.ShapeDtypeStruct((N, H), jnp.float32),
        ],
        interpret=INTERPRET,
    )(res, acc_a, acc_b, y, dinv, b.reshape(1, H), W_next)


def _layer3_body(res_ref, acca_ref, accb_ref, y_ref, dinv_ref, b_ref, r_ref):
    g = dinv_ref[...] * (acca_ref[...] + accb_ref[...] + y_ref[...]) + b_ref[...]
    r_ref[...] = jnp.maximum(res_ref[...] + g, 0.0)


def _tc_layer3(res, acc_a, acc_b, y, dinv, b):
    return pl.pallas_call(
        _layer3_body,
        grid=(N_BLKS,),
        in_specs=[
            pl.BlockSpec((ROW_BLK, H), lambda j: (j, 0)),
            pl.BlockSpec((ROW_BLK, H), lambda j: (j, 0)),
            pl.BlockSpec((ROW_BLK, H), lambda j: (j, 0)),
            pl.BlockSpec((ROW_BLK, H), lambda j: (j, 0)),
            pl.BlockSpec((ROW_BLK, 1), lambda j: (j, 0)),
            pl.BlockSpec((1, H), lambda j: (0, 0)),
        ],
        out_specs=pl.BlockSpec((ROW_BLK, H), lambda j: (j, 0)),
        out_shape=jax.ShapeDtypeStruct((N, H), jnp.float32),
        interpret=INTERPRET,
    )(res, acc_a, acc_b, y, dinv, b.reshape(1, H))


def _pool_body(batch_ref, r1_ref, r2_ref, r3_ref, ps_ref, cnt_ref):
    j = pl.program_id(0)
    b = batch_ref[0, 0, :]
    gids = lax.broadcasted_iota(jnp.int32, (G, ROW_BLK), 0)
    mask = (b[None, :] == gids).astype(jnp.float32)

    @pl.when(j == 0)
    def _():
        ps_ref[...] = jnp.zeros_like(ps_ref)
        cnt_ref[...] = jnp.zeros_like(cnt_ref)

    ps_ref[:, 0:H] += jnp.dot(mask, r1_ref[...],
                              preferred_element_type=jnp.float32)
    ps_ref[:, H:2 * H] += jnp.dot(mask, r2_ref[...],
                                  preferred_element_type=jnp.float32)
    ps_ref[:, 2 * H:3 * H] += jnp.dot(mask, r3_ref[...],
                                      preferred_element_type=jnp.float32)
    cnt_ref[...] += jnp.sum(mask, axis=1, keepdims=True)


def _tc_pool(batch3, r1, r2, r3):
    return pl.pallas_call(
        _pool_body,
        grid=(N_BLKS,),
        in_specs=[
            pl.BlockSpec((1, 1, ROW_BLK), lambda j: (j, 0, 0)),
            pl.BlockSpec((ROW_BLK, H), lambda j: (j, 0)),
            pl.BlockSpec((ROW_BLK, H), lambda j: (j, 0)),
            pl.BlockSpec((ROW_BLK, H), lambda j: (j, 0)),
        ],
        out_specs=[
            pl.BlockSpec((G, 3 * H), lambda j: (0, 0)),
            pl.BlockSpec((G, 1), lambda j: (0, 0)),
        ],
        out_shape=[
            jax.ShapeDtypeStruct((G, 3 * H), jnp.float32),
            jax.ShapeDtypeStruct((G, 1), jnp.float32),
        ],
        interpret=INTERPRET,
    )(batch3, r1, r2, r3)


def _head_body(ps_ref, cnt_ref, wp1_ref, bp1_ref, wp2_ref, bp2_ref, o_ref):
    pooled = ps_ref[...] / jnp.maximum(cnt_ref[...], 1.0)
    t = jnp.maximum(jnp.dot(pooled, wp1_ref[...],
                            preferred_element_type=jnp.float32) + bp1_ref[...],
                    0.0)
    p = jnp.dot(t, wp2_ref[...],
                preferred_element_type=jnp.float32) + bp2_ref[...]
    nrm = jnp.sqrt(jnp.sum(p * p, axis=1, keepdims=True))
    o_ref[...] = p / jnp.maximum(nrm, 1e-12)


def _tc_head(ps, cnt, Wp1, bp1, Wp2, bp2):
    return pl.pallas_call(
        _head_body,
        out_shape=jax.ShapeDtypeStruct((G, P), jnp.float32),
        interpret=INTERPRET,
    )(ps, cnt, Wp1, bp1.reshape(1, H), Wp2, bp2.reshape(1, P))


def _sc_hist(dst):
    @functools.partial(
        pl.kernel,
        out_type=jax.ShapeDtypeStruct((NW, N), jnp.float32),
        mesh=_vmesh,
        compiler_params=_sc_cp,
        scratch_types=[
            pltpu.VMEM((EH_TILE,), jnp.int32),
            pltpu.VMEM((N,), jnp.float32),
        ],
    )
    def k(dst_hbm, out_hbm, idxs, hist):
        wid = lax.axis_index("c") * NS + lax.axis_index("s")
        pltpu.sync_copy(dst_hbm.at[pl.ds(wid * EH_TILE, EH_TILE)], idxs)

        zeros = jnp.zeros((16,), jnp.float32)

        @pl.loop(0, N // 16)
        def _(i):
            hist[pl.ds(i * 16, 16)] = zeros

        ones = jnp.ones((16,), jnp.float32)

        @pl.loop(0, EH_TILE // 16)
        def _(i):
            ii = idxs[pl.ds(i * 16, 16)]
            plsc.addupdate_scatter(hist, [ii], ones)

        pltpu.sync_copy(hist, out_hbm.at[wid])

    return k(dst)


def _sc_scatter(y, srcp, dstp):
    @functools.partial(
        pl.kernel,
        out_type=jax.ShapeDtypeStruct((NC, NPAD, H), jnp.float32),
        mesh=_vmesh,
        compiler_params=_sc_cp,
        scratch_types=[
            pltpu.VMEM((IB, CHUNK), jnp.int32),
            pltpu.VMEM((IB, CHUNK), jnp.int32),
            pltpu.VMEM((IB, CHUNK), jnp.int32),
            pltpu.VMEM((IB, CHUNK), jnp.int32),
            pltpu.VMEM((CHUNK,), jnp.int32),
            pltpu.VMEM((CHUNK,), jnp.int32),
            pltpu.VMEM((CHUNK, H), jnp.float32),
            pltpu.VMEM((CHUNK, H), jnp.float32),
            pltpu.VMEM_SHARED((NPAD, H), jnp.float32),
            pltpu.SemaphoreType.DMA,
            pltpu.SemaphoreType.DMA,
            pltpu.SemaphoreType.DMA,
        ],
    )
    def k(y_hbm, src_hbm, dst_hbm, out_hbm, s0, d0, s1, d1, sidx, didx,
          rows0, rows1, acc, gsem, ssem, isem):
        cid = lax.axis_index("c")
        sid = lax.axis_index("s")

        zeros = jnp.zeros((16,), jnp.float32)

        @pl.loop(0, CHUNK)
        def _(r):
            @pl.loop(0, H // 16)
            def _(q):
                rows0[r, pl.ds(q * 16, 16)] = zeros

        @pl.loop(0, ROWS_TILE // CHUNK)
        def _(z):
            pltpu.sync_copy(
                rows0, acc.at[pl.ds(sid * ROWS_TILE + z * CHUNK, CHUNK)])

        plsc.subcore_barrier()

        @pl.when(cid == 0)
        def _():
            base_row = sid * NCH0

            def i_start(b, sb, db):
                pltpu.async_copy(src_hbm.at[pl.ds(base_row + b * IB, IB)],
                                 sb, isem)
                pltpu.async_copy(dst_hbm.at[pl.ds(base_row + b * IB, IB)],
                                 db, isem)

            def i_wait(b, sb, db):
                pltpu.make_async_copy(
                    src_hbm.at[pl.ds(base_row + b * IB, IB)], sb, isem).wait()
                pltpu.make_async_copy(
                    dst_hbm.at[pl.ds(base_row + b * IB, IB)], db, isem).wait()

            def g_start(sb, j, buf):
                pltpu.async_copy(y_hbm.at[sb.at[j]], buf, gsem)

            def g_wait(sb, j, buf):
                pltpu.make_async_copy(y_hbm.at[sb.at[j]], buf, gsem).wait()

            def s_start(db, j, buf):
                pltpu.async_copy(buf, acc.at[db.at[j]], ssem, add=True)

            def s_wait(db, j, buf):
                pltpu.make_async_copy(buf, acc.at[db.at[j]], ssem).wait()

            def do_block(b, sb, db, sb_next, db_next):
                @pl.when(b + 1 < NB0)
                def _():
                    i_start(b + 1, sb_next, db_next)

                for j in range(0, IB, 2):
                    g_start(sb, j + 1, rows1)
                    g_wait(sb, j, rows0)
                    s_start(db, j, rows0)
                    g_wait(sb, j + 1, rows1)
                    s_wait(db, j, rows0)
                    if j + 2 < IB:
                        g_start(sb, j + 2, rows0)
                    else:
                        @pl.when(b + 1 < NB0)
                        def _():
                            i_wait(b + 1, sb_next, db_next)
                            g_start(sb_next, 0, rows0)

                    s_start(db, j + 1, rows1)
                    s_wait(db, j + 1, rows1)

            i_start(0, s0, d0)
            i_wait(0, s0, d0)
            g_start(s0, 0, rows0)

            @pl.loop(0, NB0 // 2)
            def _(p):
                do_block(2 * p, s0, d0, s1, d1)
                do_block(2 * p + 1, s1, d1, s0, d0)

        @pl.when(cid == 1)
        def _():
            base_row = NS * NCH0 + sid * NCH1

            @pl.loop(0, NCH1)
            def _(cnk):
                r = base_row + cnk
                pltpu.sync_copy(src_hbm.at[r], sidx)
                pltpu.sync_copy(dst_hbm.at[r], didx)
                pltpu.async_copy(y_hbm.at[sidx], rows0, gsem).wait()
                pltpu.sync_copy(rows0, acc.at[didx], add=True)

        plsc.subcore_barrier()

        pltpu.sync_copy(acc.at[pl.ds(sid * ROWS_TILE, ROWS_TILE)],
                        out_hbm.at[cid, pl.ds(sid * ROWS_TILE, ROWS_TILE)])

    out = k(y, srcp, dstp)
    return out[0, :N], out[1, :N]


def kernel(x, edge_index, batch, W_in, b_in, W1, b1, W2, b2, W3, b3,
           Wp1, bp1, Wp2, bp2):
    src, dst = edge_index[0], edge_index[1]

    srcp = jnp.concatenate([src, jnp.zeros(EPAD - E, jnp.int32)])
    dstp = jnp.concatenate([dst, jnp.full(EPAD - E, NPAD - 1, jnp.int32)])
    srcp = srcp.reshape(EPAD // CHUNK, CHUNK)
    dstp = dstp.reshape(EPAD // CHUNK, CHUNK)

    hist = _sc_hist(dst)
    dinv = _tc_dinv(hist)

    h, y1 = _tc_in(x, W_in, b_in, W1, dinv)
    a1, a1b = _sc_scatter(y1, srcp, dstp)
    r1, y2 = _tc_layer(h, a1, a1b, y1, dinv, b1, W2)
    a2, a2b = _sc_scatter(y2, srcp, dstp)
    r2, y3 = _tc_layer(r1, a2, a2b, y2, dinv, b2, W3)
    a3, a3b = _sc_scatter(y3, srcp, dstp)
    r3 = _tc_layer3(r2, a3, a3b, y3, dinv, b3)

    batch3 = batch.reshape(N_BLKS, 1, ROW_BLK)
    ps, cnt = _tc_pool(batch3, r1, r2, r3)
    return _tc_head(ps, cnt, Wp1, bp1, Wp2, bp2)


# speedup vs baseline: 2.1289x; 1.1822x over previous
import functools

import jax
import jax.numpy as jnp
from jax import lax
from jax.experimental import pallas as pl
from jax.experimental.pallas import tpu as pltpu
from jax.experimental.pallas import tpu_sc as plsc

N = 10000
E = 320000
D = 128
H = 128
P = 128
G = 64

INTERPRET = False

ROW_BLK = 1000
N_BLKS = N // ROW_BLK

NC = 2
NS = 16
NW = NC * NS

CHUNK = 128
IB = 8
EPAD = -(-E // (NW * CHUNK)) * (NW * CHUNK)
TOT_CHUNKS = EPAD // (NS * CHUNK)
NCH0 = 144
NB0 = NCH0 // IB
NCH1 = TOT_CHUNKS - NCH0
NPAD = 10240
ROWS_TILE = NPAD // NS
EH_TILE = E // NW

_vmesh = plsc.VectorSubcoreMesh(core_axis_name="c", subcore_axis_name="s")

import dataclasses as _dc

_sc_cp = pltpu.CompilerParams()
if "needs_layout_passes" in pltpu.CompilerParams.__dataclass_fields__:
    _sc_cp = _dc.replace(_sc_cp, needs_layout_passes=False)


def _dinv_body(hist_ref, o_ref):
    deg = jnp.sum(hist_ref[...], axis=0) + 1.0
    o_ref[...] = lax.rsqrt(deg)[:, None]


def _tc_dinv(hist):
    return pl.pallas_call(
        _dinv_body,
        out_shape=jax.ShapeDtypeStruct((N, 1), jnp.float32),
        interpret=INTERPRET,
    )(hist)


def _in_body(x_ref, win_ref, bin_ref, w1_ref, dinv_ref, h_ref, y_ref):
    h = jnp.dot(x_ref[...], win_ref[...],
                preferred_element_type=jnp.float32) + bin_ref[...]
    h_ref[...] = h
    y_ref[...] = dinv_ref[...] * jnp.dot(h, w1_ref[...],
                                         preferred_element_type=jnp.float32)


def _tc_in(x, W_in, b_in, W1, dinv):
    return pl.pallas_call(
        _in_body,
        grid=(N_BLKS,),
        in_specs=[
            pl.BlockSpec((ROW_BLK, D), lambda j: (j, 0)),
            pl.BlockSpec((D, H), lambda j: (0, 0)),
            pl.BlockSpec((1, H), lambda j: (0, 0)),
            pl.BlockSpec((H, H), lambda j: (0, 0)),
            pl.BlockSpec((ROW_BLK, 1), lambda j: (j, 0)),
        ],
        out_specs=[
            pl.BlockSpec((ROW_BLK, H), lambda j: (j, 0)),
            pl.BlockSpec((ROW_BLK, H), lambda j: (j, 0)),
        ],
        out_shape=[
            jax.ShapeDtypeStruct((N, H), jnp.float32),
            jax.ShapeDtypeStruct((N, H), jnp.float32),
        ],
        interpret=INTERPRET,
    )(x, W_in, b_in.reshape(1, H), W1, dinv)


def _layer_body(res_ref, acca_ref, accb_ref, y_ref, dinv_ref, b_ref, wn_ref,
                r_ref, yn_ref):
    dinv = dinv_ref[...]
    g = dinv * (acca_ref[...] + accb_ref[...] + y_ref[...]) + b_ref[...]
    r = jnp.maximum(res_ref[...] + g, 0.0)
    r_ref[...] = r
    yn_ref[...] = dinv * jnp.dot(r, wn_ref[...],
                                 preferred_element_type=jnp.float32)


def _tc_layer(res, acc_a, acc_b, y, dinv, b, W_next):
    return pl.pallas_call(
        _layer_body,
        grid=(N_BLKS,),
        in_specs=[
            pl.BlockSpec((ROW_BLK, H), lambda j: (j, 0)),
            pl.BlockSpec((ROW_BLK, H), lambda j: (j, 0)),
            pl.BlockSpec((ROW_BLK, H), lambda j: (j, 0)),
            pl.BlockSpec((ROW_BLK, H), lambda j: (j, 0)),
            pl.BlockSpec((ROW_BLK, 1), lambda j: (j, 0)),
            pl.BlockSpec((1, H), lambda j: (0, 0)),
            pl.BlockSpec((H, H), lambda j: (0, 0)),
        ],
        out_specs=[
            pl.BlockSpec((ROW_BLK, H), lambda j: (j, 0)),
            pl.BlockSpec((ROW_BLK, H), lambda j: (j, 0)),
        ],
        out_shape=[
            jax.ShapeDtypeStruct((N, H), jnp.float32),
            jax.ShapeDtypeStruct((N, H), jnp.float32),
        ],
        interpret=INTERPRET,
    )(res, acc_a, acc_b, y, dinv, b.reshape(1, H), W_next)


def _layer3_body(res_ref, acca_ref, accb_ref, y_ref, dinv_ref, b_ref, r_ref):
    g = dinv_ref[...] * (acca_ref[...] + accb_ref[...] + y_ref[...]) + b_ref[...]
    r_ref[...] = jnp.maximum(res_ref[...] + g, 0.0)


def _tc_layer3(res, acc_a, acc_b, y, dinv, b):
    return pl.pallas_call(
        _layer3_body,
        grid=(N_BLKS,),
        in_specs=[
            pl.BlockSpec((ROW_BLK, H), lambda j: (j, 0)),
            pl.BlockSpec((ROW_BLK, H), lambda j: (j, 0)),
            pl.BlockSpec((ROW_BLK, H), lambda j: (j, 0)),
            pl.BlockSpec((ROW_BLK, H), lambda j: (j, 0)),
            pl.BlockSpec((ROW_BLK, 1), lambda j: (j, 0)),
            pl.BlockSpec((1, H), lambda j: (0, 0)),
        ],
        out_specs=pl.BlockSpec((ROW_BLK, H), lambda j: (j, 0)),
        out_shape=jax.ShapeDtypeStruct((N, H), jnp.float32),
        interpret=INTERPRET,
    )(res, acc_a, acc_b, y, dinv, b.reshape(1, H))


def _pool_body(batch_ref, r1_ref, r2_ref, r3_ref, ps_ref, cnt_ref):
    j = pl.program_id(0)
    b = batch_ref[0, 0, :]
    gids = lax.broadcasted_iota(jnp.int32, (G, ROW_BLK), 0)
    mask = (b[None, :] == gids).astype(jnp.float32)

    @pl.when(j == 0)
    def _():
        ps_ref[...] = jnp.zeros_like(ps_ref)
        cnt_ref[...] = jnp.zeros_like(cnt_ref)

    ps_ref[:, 0:H] += jnp.dot(mask, r1_ref[...],
                              preferred_element_type=jnp.float32)
    ps_ref[:, H:2 * H] += jnp.dot(mask, r2_ref[...],
                                  preferred_element_type=jnp.float32)
    ps_ref[:, 2 * H:3 * H] += jnp.dot(mask, r3_ref[...],
                                      preferred_element_type=jnp.float32)
    cnt_ref[...] += jnp.sum(mask, axis=1, keepdims=True)


def _tc_pool(batch3, r1, r2, r3):
    return pl.pallas_call(
        _pool_body,
        grid=(N_BLKS,),
        in_specs=[
            pl.BlockSpec((1, 1, ROW_BLK), lambda j: (j, 0, 0)),
            pl.BlockSpec((ROW_BLK, H), lambda j: (j, 0)),
            pl.BlockSpec((ROW_BLK, H), lambda j: (j, 0)),
            pl.BlockSpec((ROW_BLK, H), lambda j: (j, 0)),
        ],
        out_specs=[
            pl.BlockSpec((G, 3 * H), lambda j: (0, 0)),
            pl.BlockSpec((G, 1), lambda j: (0, 0)),
        ],
        out_shape=[
            jax.ShapeDtypeStruct((G, 3 * H), jnp.float32),
            jax.ShapeDtypeStruct((G, 1), jnp.float32),
        ],
        interpret=INTERPRET,
    )(batch3, r1, r2, r3)


def _head_body(ps_ref, cnt_ref, wp1_ref, bp1_ref, wp2_ref, bp2_ref, o_ref):
    pooled = ps_ref[...] / jnp.maximum(cnt_ref[...], 1.0)
    t = jnp.maximum(jnp.dot(pooled, wp1_ref[...],
                            preferred_element_type=jnp.float32) + bp1_ref[...],
                    0.0)
    p = jnp.dot(t, wp2_ref[...],
                preferred_element_type=jnp.float32) + bp2_ref[...]
    nrm = jnp.sqrt(jnp.sum(p * p, axis=1, keepdims=True))
    o_ref[...] = p / jnp.maximum(nrm, 1e-12)


def _tc_head(ps, cnt, Wp1, bp1, Wp2, bp2):
    return pl.pallas_call(
        _head_body,
        out_shape=jax.ShapeDtypeStruct((G, P), jnp.float32),
        interpret=INTERPRET,
    )(ps, cnt, Wp1, bp1.reshape(1, H), Wp2, bp2.reshape(1, P))


def _sc_hist(dst):
    @functools.partial(
        pl.kernel,
        out_type=jax.ShapeDtypeStruct((NW, N), jnp.float32),
        mesh=_vmesh,
        compiler_params=_sc_cp,
        scratch_types=[
            pltpu.VMEM((EH_TILE,), jnp.int32),
            pltpu.VMEM((N,), jnp.float32),
        ],
    )
    def k(dst_hbm, out_hbm, idxs, hist):
        wid = lax.axis_index("c") * NS + lax.axis_index("s")
        pltpu.sync_copy(dst_hbm.at[pl.ds(wid * EH_TILE, EH_TILE)], idxs)

        zeros = jnp.zeros((16,), jnp.float32)

        @pl.loop(0, N // 16)
        def _(i):
            hist[pl.ds(i * 16, 16)] = zeros

        ones = jnp.ones((16,), jnp.float32)

        @pl.loop(0, EH_TILE // 16)
        def _(i):
            ii = idxs[pl.ds(i * 16, 16)]
            plsc.addupdate_scatter(hist, [ii], ones)

        pltpu.sync_copy(hist, out_hbm.at[wid])

    return k(dst)


def _sc_scatter(y, srcp, dstp):
    @functools.partial(
        pl.kernel,
        out_type=jax.ShapeDtypeStruct((NC, NPAD, H), jnp.float32),
        mesh=_vmesh,
        compiler_params=_sc_cp,
        scratch_types=[
            pltpu.VMEM((IB, CHUNK), jnp.int32),
            pltpu.VMEM((IB, CHUNK), jnp.int32),
            pltpu.VMEM((IB, CHUNK), jnp.int32),
            pltpu.VMEM((IB, CHUNK), jnp.int32),
            pltpu.VMEM((CHUNK,), jnp.int32),
            pltpu.VMEM((CHUNK,), jnp.int32),
            pltpu.VMEM((CHUNK, H), jnp.float32),
            pltpu.VMEM((CHUNK, H), jnp.float32),
            pltpu.VMEM_SHARED((NPAD, H), jnp.float32),
            pltpu.SemaphoreType.DMA,
            pltpu.SemaphoreType.DMA,
            pltpu.SemaphoreType.DMA,
        ],
    )
    def k(y_hbm, src_hbm, dst_hbm, out_hbm, s0, d0, s1, d1, sidx, didx,
          rows0, rows1, acc, gsem, ssem, isem):
        cid = lax.axis_index("c")
        sid = lax.axis_index("s")

        zeros = jnp.zeros((16,), jnp.float32)

        @pl.loop(0, CHUNK)
        def _(r):
            @pl.loop(0, H // 16)
            def _(q):
                rows0[r, pl.ds(q * 16, 16)] = zeros

        @pl.loop(0, ROWS_TILE // CHUNK)
        def _(z):
            pltpu.sync_copy(
                rows0, acc.at[pl.ds(sid * ROWS_TILE + z * CHUNK, CHUNK)])

        plsc.subcore_barrier()

        @pl.when(cid == 0)
        def _():
            base_row = sid * NCH0

            def i_start(b, sb, db):
                pltpu.async_copy(src_hbm.at[pl.ds(base_row + b * IB, IB)],
                                 sb, isem)
                pltpu.async_copy(dst_hbm.at[pl.ds(base_row + b * IB, IB)],
                                 db, isem)

            def i_wait(b, sb, db):
                pltpu.make_async_copy(
                    src_hbm.at[pl.ds(base_row + b * IB, IB)], sb, isem).wait()
                pltpu.make_async_copy(
                    dst_hbm.at[pl.ds(base_row + b * IB, IB)], db, isem).wait()

            def g_start(sb, j, buf):
                pltpu.async_copy(y_hbm.at[sb.at[j]], buf, gsem)

            def g_wait(sb, j, buf):
                pltpu.make_async_copy(y_hbm.at[sb.at[j]], buf, gsem).wait()

            def s_start(db, j, buf):
                pltpu.async_copy(buf, acc.at[db.at[j]], ssem, add=True)

            def s_wait(db, j, buf):
                pltpu.make_async_copy(buf, acc.at[db.at[j]], ssem).wait()

            def do_block(b, sb, db, sb_next, db_next):
                @pl.when(b + 1 < NB0)
                def _():
                    i_start(b + 1, sb_next, db_next)

                for j in range(0, IB, 2):
                    g_start(sb, j + 1, rows1)
                    g_wait(sb, j, rows0)
                    s_start(db, j, rows0)
                    g_wait(sb, j + 1, rows1)
                    s_wait(db, j, rows0)
                    if j + 2 < IB:
                        g_start(sb, j + 2, rows0)
                    else:
                        @pl.when(b + 1 < NB0)
                        def _():
                            i_wait(b + 1, sb_next, db_next)
                            g_start(sb_next, 0, rows0)

                    s_start(db, j + 1, rows1)
                    s_wait(db, j + 1, rows1)

            i_start(0, s0, d0)
            i_wait(0, s0, d0)
            g_start(s0, 0, rows0)

            @pl.loop(0, NB0 // 2)
            def _(p):
                do_block(2 * p, s0, d0, s1, d1)
                do_block(2 * p + 1, s1, d1, s0, d0)

        @pl.when(cid == 1)
        def _():
            base_row = NS * NCH0 + sid * NCH1

            @pl.loop(0, NCH1)
            def _(cnk):
                r = base_row + cnk
                pltpu.sync_copy(src_hbm.at[r], sidx)
                pltpu.sync_copy(dst_hbm.at[r], didx)
                pltpu.async_copy(y_hbm.at[sidx], rows0, gsem).wait()
                pltpu.sync_copy(rows0, acc.at[didx], add=True)

        plsc.subcore_barrier()

        pltpu.sync_copy(acc.at[pl.ds(sid * ROWS_TILE, ROWS_TILE)],
                        out_hbm.at[cid, pl.ds(sid * ROWS_TILE, ROWS_TILE)])

    out = k(y, srcp, dstp)
    return out[0, :N], out[1, :N]


def kernel(x, edge_index, batch, W_in, b_in, W1, b1, W2, b2, W3, b3,
           Wp1, bp1, Wp2, bp2):
    src, dst = edge_index[0], edge_index[1]

    srcp = jnp.concatenate([src, jnp.zeros(EPAD - E, jnp.int32)])
    dstp = jnp.concatenate([dst, jnp.full(EPAD - E, NPAD - 1, jnp.int32)])
    srcp = srcp.reshape(EPAD // CHUNK, CHUNK)
    dstp = dstp.reshape(EPAD // CHUNK, CHUNK)

    hist = _sc_hist(dst)
    dinv = _tc_dinv(hist)

    h, y1 = _tc_in(x, W_in, b_in, W1, dinv)
    a1, a1b = _sc_scatter(y1, srcp, dstp)
    r1, y2 = _tc_layer(h, a1, a1b, y1, dinv, b1, W2)
    a2, a2b = _sc_scatter(y2, srcp, dstp)
    r2, y3 = _tc_layer(r1, a2, a2b, y2, dinv, b2, W3)
    a3, a3b = _sc_scatter(y3, srcp, dstp)
    r3 = _tc_layer3(r2, a3, a3b, y3, dinv, b3)

    batch3 = batch.reshape(N_BLKS, 1, ROW_BLK)
    ps, cnt = _tc_pool(batch3, r1, r2, r3)
    return _tc_head(ps, cnt, Wp1, bp1, Wp2, bp2)
